# baseline (speedup 1.0000x reference)
import dataclasses
import functools

import jax
import jax.numpy as jnp
from jax import lax
from jax.experimental import pallas as pl
from jax.experimental.pallas import tpu as pltpu
from jax.experimental.pallas import tpu_sc as plsc

N = 10000
E = 320000
DIN = 128
F = 512
NCLS = 40
NG = 64
NC = 2
NS = 16
NW = NC * NS
CE = 256
NCK = E // CE
TMAX = -(-NCK // NW)
RU = 80
NU = N // RU
UTMAX = -(-NU // NS)
AROWS = N * 8 // 128
NBLK = 10
BN = N // NBLK

_f32 = jnp.float32
_i32 = jnp.int32


def _k1_common(xb, w_ref, p_ref, h0, h1, h2, h3, aall, m, i):
    h = jnp.dot(xb, w_ref[...], preferred_element_type=_f32)
    a = jnp.dot(h, p_ref[...], preferred_element_type=_f32)
    for k, href in enumerate((h0, h1, h2, h3)):
        href[...] = h[:, k * 128:(k + 1) * 128]
    aall[...] = a
    bm = jnp.broadcast_to(jnp.max(a, axis=0, keepdims=True), (8, 16))

    @pl.when(i == 0)
    def _():
        m[...] = bm

    @pl.when(i != 0)
    def _():
        m[...] = jnp.maximum(m[...], bm)


def _k1a_body(x_ref, w_ref, p_ref, h0, h1, h2, h3, aall, m):
    _k1_common(x_ref[...], w_ref, p_ref, h0, h1, h2, h3, aall, m,
               pl.program_id(0))


def _k1b_body(o0, o1, o2, o3, b_ref, w_ref, p_ref, h0, h1, h2, h3, aall, m):
    parts = []
    for k, oref in enumerate((o0, o1, o2, o3)):
        ob = oref[...]
        parts.append(jnp.maximum(ob[0] + ob[1] + b_ref[:, k * 128:(k + 1) * 128], 0.0))
    _k1_common(jnp.concatenate(parts, axis=1), w_ref, p_ref,
               h0, h1, h2, h3, aall, m, pl.program_id(0))


def _k1_outs():
    shapes = [jax.ShapeDtypeStruct((N, 128), _f32) for _ in range(4)]
    shapes += [jax.ShapeDtypeStruct((N, 16), _f32),
               jax.ShapeDtypeStruct((8, 16), _f32)]
    specs = [pl.BlockSpec((BN, 128), lambda i: (i, 0)) for _ in range(4)]
    specs += [pl.BlockSpec((BN, 16), lambda i: (i, 0)),
              pl.BlockSpec((8, 16), lambda i: (0, 0))]
    return shapes, specs


def _tc_layer1(x, w, p):
    shapes, specs = _k1_outs()
    return pl.pallas_call(
        _k1a_body,
        grid=(NBLK,),
        in_specs=[
            pl.BlockSpec((BN, DIN), lambda i: (i, 0)),
            pl.BlockSpec((DIN, F), lambda i: (0, 0)),
            pl.BlockSpec((F, 16), lambda i: (0, 0)),
        ],
        out_specs=specs,
        out_shape=shapes,
    )(x, w, p)


def _tc_layer2(o_parts, bias_r, w, p):
    shapes, specs = _k1_outs()
    return pl.pallas_call(
        _k1b_body,
        grid=(NBLK,),
        in_specs=[pl.BlockSpec((2, BN, 128), lambda i: (0, i, 0))] * 4 + [
            pl.BlockSpec((1, F), lambda i: (0, 0)),
            pl.BlockSpec((F, F), lambda i: (0, 0)),
            pl.BlockSpec((F, 16), lambda i: (0, 0)),
        ],
        out_specs=specs,
        out_shape=shapes,
    )(*o_parts, bias_r, w, p)


def _den_body(d_ref, o_ref):
    o_ref[...] = jnp.sum(d_ref[...], axis=0) + 1e-16


def _tc_den(dpart):
    return pl.pallas_call(
        _den_body,
        out_shape=jax.ShapeDtypeStruct((AROWS, 128), _f32),
    )(dpart)


def _pool_body(o0, o1, o2, o3, b_ref, bat_ref, lw_ref, lb_ref, probs_ref,
               pool_scr, cnt_scr):
    i = pl.program_id(0)

    @pl.when(i == 0)
    def _():
        pool_scr[...] = jnp.zeros((NG, F), _f32)
        cnt_scr[...] = jnp.zeros((NG, 128), _f32)

    parts = []
    for k, oref in enumerate((o0, o1, o2, o3)):
        ob = oref[...]
        parts.append(jnp.maximum(ob[0] + ob[1] + b_ref[:, k * 128:(k + 1) * 128], 0.0))
    h2 = jnp.concatenate(parts, axis=1)
    bb = bat_ref[0]
    oh = (lax.broadcasted_iota(_i32, (NG, BN), 0) == bb).astype(_f32)
    pool_scr[...] += jnp.dot(oh, h2, preferred_element_type=_f32)
    cnt_scr[...] += jnp.broadcast_to(jnp.sum(oh, axis=1, keepdims=True), (NG, 128))

    @pl.when(i == NBLK - 1)
    def _():
        cnt = jnp.maximum(cnt_scr[:, :1], 1.0)
        pooled = pool_scr[...] / cnt
        logits = jnp.dot(pooled, lw_ref[...], preferred_element_type=_f32) + lb_ref[...]
        mx = jnp.max(logits, axis=1, keepdims=True)
        e = jnp.exp(logits - mx)
        probs_ref[...] = e / jnp.sum(e, axis=1, keepdims=True)


def _tc_pool(o_parts, bias_r, batch3, lin_w, lin_br):
    return pl.pallas_call(
        _pool_body,
        grid=(NBLK,),
        in_specs=[pl.BlockSpec((2, BN, 128), lambda i: (0, i, 0))] * 4 + [
            pl.BlockSpec((1, F), lambda i: (0, 0)),
            pl.BlockSpec((1, 1, BN), lambda i: (i, 0, 0)),
            pl.BlockSpec((F, NCLS), lambda i: (0, 0)),
            pl.BlockSpec((1, NCLS), lambda i: (0, 0)),
        ],
        out_specs=pl.BlockSpec((NG, NCLS), lambda i: (0, 0)),
        out_shape=jax.ShapeDtypeStruct((NG, NCLS), _f32),
        scratch_shapes=[
            pltpu.VMEM((NG, F), _f32),
            pltpu.VMEM((NG, 128), _f32),
        ],
    )(*o_parts, bias_r, batch3, lin_w, lin_br)


_MESH = plsc.VectorSubcoreMesh(core_axis_name="c", subcore_axis_name="s")
_IOTA = lambda: lax.iota(_i32, 16)

_SC_PARAMS = pltpu.CompilerParams()
if "needs_layout_passes" in pltpu.CompilerParams.__dataclass_fields__:
    _SC_PARAMS = dataclasses.replace(_SC_PARAMS, needs_layout_passes=False)


def _worker_chunks(w, body):
    @pl.loop(0, TMAX)
    def _(t):
        cid = w + t * NW

        @pl.when(cid < NCK)
        def _():
            body(cid)


def _sc_alpha_src(asrc_t, src3):
    @functools.partial(
        pl.kernel,
        out_type=jax.ShapeDtypeStruct((NCK, 16, 128), _f32),
        mesh=_MESH,
        compiler_params=_SC_PARAMS,
        scratch_types=[
            pltpu.VMEM((AROWS, 128), _f32),
            pltpu.VMEM((2, 128), _i32),
            pltpu.VMEM((16, 128), _f32),
        ],
    )
    def k(at_h, src_h, out_h, atab, srcv, obuf):
        c = lax.axis_index("c")
        s = lax.axis_index("s")
        w = c * NS + s
        pltpu.sync_copy(at_h, atab)

        def chunk(cid):
            pltpu.sync_copy(src_h.at[cid], srcv)
            for g in range(16):
                sv = srcv[g // 8, pl.ds((g % 8) * 16, 16)]
                for h in range(8):
                    idx = sv * 8 + h
                    v = plsc.load_gather(
                        atab, [lax.shift_right_logical(idx, 7), idx & 127])
                    obuf[g, pl.ds(h * 16, 16)] = v
            pltpu.sync_copy(obuf, out_h.at[cid])

        _worker_chunks(w, chunk)

    return k(asrc_t, src3)


def _sc_alpha_dst_exp(adst_t, m, alphaT, dst3):
    @functools.partial(
        pl.kernel,
        out_type=jax.ShapeDtypeStruct((NCK, 16, 128), _f32),
        mesh=_MESH,
        compiler_params=_SC_PARAMS,
        scratch_types=[
            pltpu.VMEM((AROWS, 128), _f32),
            pltpu.VMEM((2, 128), _i32),
            pltpu.VMEM((16, 128), _f32),
            pltpu.VMEM((16, 128), _f32),
            pltpu.VMEM((8, 16), _f32),
        ],
    )
    def k(at_h, m_h, al_h, dst_h, out_h, atab, dstv, abuf, ebuf, mb):
        c = lax.axis_index("c")
        s = lax.axis_index("s")
        w = c * NS + s
        pltpu.sync_copy(at_h, atab)
        pltpu.sync_copy(m_h, mb)
        msp = []
        for h in range(8):
            ms = plsc.load_gather(mb, [jnp.full((16,), 0, _i32),
                                       jnp.full((16,), h, _i32)])
            md = plsc.load_gather(mb, [jnp.full((16,), 0, _i32),
                                       jnp.full((16,), h + 8, _i32)])
            msp.append(ms + md)

        def chunk(cid):
            pltpu.sync_copy(dst_h.at[cid], dstv)
            pltpu.sync_copy(al_h.at[cid], abuf)
            for g in range(16):
                dv = dstv[g // 8, pl.ds((g % 8) * 16, 16)]
                for h in range(8):
                    idx = dv * 8 + h
                    d = plsc.load_gather(
                        atab, [lax.shift_right_logical(idx, 7), idx & 127])
                    a = abuf[g, pl.ds(h * 16, 16)] + d
                    al = jnp.maximum(a, 0.2 * a) - msp[h]
                    ebuf[g, pl.ds(h * 16, 16)] = jnp.exp(al)
            pltpu.sync_copy(ebuf, out_h.at[cid])

        _worker_chunks(w, chunk)

    return k(adst_t, m, alphaT, dst3)


def _sc_denom(exT, dst3, z625):
    @functools.partial(
        pl.kernel,
        out_type=jax.ShapeDtypeStruct((NW, AROWS, 128), _f32),
        mesh=_MESH,
        compiler_params=_SC_PARAMS,
        scratch_types=[
            pltpu.VMEM((AROWS, 128), _f32),
            pltpu.VMEM((2, 128), _i32),
            pltpu.VMEM((16, 128), _f32),
        ],
    )
    def k(ex_h, dst_h, z_h, out_h, den, dstv, ebuf):
        c = lax.axis_index("c")
        s = lax.axis_index("s")
        w = c * NS + s
        pltpu.sync_copy(z_h, den)
        lanes = _IOTA()
        mask = lanes < 8
        hsel = lanes & 7

        def chunk(cid):
            pltpu.sync_copy(dst_h.at[cid], dstv)
            pltpu.sync_copy(ex_h.at[cid], ebuf)

            @pl.loop(0, CE)
            def _(i):
                dsp = plsc.load_gather(
                    dstv, [jnp.full((16,), lax.shift_right_logical(i, 7), _i32),
                           jnp.full((16,), i & 127, _i32)])
                exr = plsc.load_gather(
                    ebuf, [jnp.full((16,), lax.shift_right_logical(i, 4), _i32),
                           (hsel * 16 + (i & 15))])
                idx = dsp * 8 + hsel
                plsc.addupdate_scatter(
                    den, [lax.shift_right_logical(idx, 7), idx & 127],
                    exr, mask=mask)

        _worker_chunks(w, chunk)
        pltpu.sync_copy(den, out_h.at[w])

    return k(exT, dst3, z625)


def _sc_coef(den_t, exT, dst3):
    @functools.partial(
        pl.kernel,
        out_type=jax.ShapeDtypeStruct((NCK, 16, 128), _f32),
        mesh=_MESH,
        compiler_params=_SC_PARAMS,
        scratch_types=[
            pltpu.VMEM((AROWS, 128), _f32),
            pltpu.VMEM((2, 128), _i32),
            pltpu.VMEM((16, 128), _f32),
            pltpu.VMEM((16, 128), _f32),
        ],
    )
    def k(dt_h, ex_h, dst_h, out_h, dtab, dstv, ebuf, cbuf):
        c = lax.axis_index("c")
        s = lax.axis_index("s")
        w = c * NS + s
        pltpu.sync_copy(dt_h, dtab)

        def chunk(cid):
            pltpu.sync_copy(dst_h.at[cid], dstv)
            pltpu.sync_copy(ex_h.at[cid], ebuf)
            for g in range(16):
                dv = dstv[g // 8, pl.ds((g % 8) * 16, 16)]
                for h in range(8):
                    idx = dv * 8 + h
                    den = plsc.load_gather(
                        dtab, [lax.shift_right_logical(idx, 7), idx & 127])
                    cbuf[g, pl.ds(h * 16, 16)] = \
                        ebuf[g, pl.ds(h * 16, 16)] / den
            pltpu.sync_copy(cbuf, out_h.at[cid])

        _worker_chunks(w, chunk)

    return k(den_t, exT, dst3)


CB = 128
BCK = E // CB
BSLOT = 78


def _sc_message(e2, coefT2, hcols, z128):
    @functools.partial(
        pl.kernel,
        out_type=[jax.ShapeDtypeStruct((2, N, 128), _f32) for _ in range(4)],
        mesh=_MESH,
        compiler_params=_SC_PARAMS,
        scratch_types=[
            pltpu.VMEM((2, 2, 128), _i32),
            pltpu.VMEM((2, 8, 128), _f32),
            pltpu.VMEM((2, CB, 128), _f32),
            pltpu.VMEM_SHARED((N, 128), _f32),
            pltpu.SemaphoreType.DMA,
            pltpu.SemaphoreType.DMA,
            pltpu.SemaphoreType.DMA,
            pltpu.SemaphoreType.DMA,
            pltpu.SemaphoreType.DMA,
            pltpu.SemaphoreType.DMA,
        ],
    )
    def k(e2_h, co_h, h0_h, h1_h, h2_h, h3_h, z_h,
          o0_h, o1_h, o2_h, o3_h,
          ebuf, cbuf, hbuf, shout, si0, si1, sc0, sc1, sh0, sh1):
        c = lax.axis_index("c")
        s = lax.axis_index("s")
        w = c * NS + s
        h_refs = (h0_h, h1_h, h2_h, h3_h)
        o_refs = (o0_h, o1_h, o2_h, o3_h)
        si = (si0, si1)
        sc = (sc0, sc1)
        sh = (sh0, sh1)

        def cid_of(t):
            return w + t * NW

        def ic_cp(cid, b):
            return (pltpu.make_async_copy(e2_h.at[cid], ebuf.at[b], si[b]),
                    pltpu.make_async_copy(co_h.at[cid], cbuf.at[b], sc[b]))

        def ic_start(cid, b):
            for cp in ic_cp(cid, b):
                cp.start()

        def ic_wait(cid, b):
            for cp in ic_cp(cid, b):
                cp.wait()

        for kcol in range(4):
            h_h = h_refs[kcol]
            o_h = o_refs[kcol]

            def h_cp(b):
                return pltpu.make_async_copy(
                    h_h.at[ebuf.at[b, 0]], hbuf.at[b], sh[b])

            def compute(b):
                @pl.loop(0, 8)
                def _(g):
                    gsp = jnp.full((16,), g, _i32)

                    @pl.loop(0, 16)
                    def _(l):
                        c0 = jnp.full((16,), 32 * kcol + l, _i32)
                        b0 = plsc.load_gather(cbuf.at[b], [gsp, c0])
                        b1 = plsc.load_gather(cbuf.at[b], [gsp, c0 + 16])
                        i = g * 16 + l
                        for j2 in range(8):
                            sel = b0 if j2 < 4 else b1
                            sl = pl.ds(j2 * 16, 16)
                            hbuf[b, i, sl] = hbuf[b, i, sl] * sel

            def scatter(b):
                pltpu.sync_copy(hbuf.at[b], shout.at[ebuf.at[b, 1]], add=True)

            def step(t, b, refill):
                h_cp(b).wait()
                ic_wait(cid_of(t + 1), 1 - b)
                h_cp(1 - b).start()
                compute(b)
                scatter(b)
                if refill:
                    ic_start(cid_of(t + 2), b)

            @pl.loop(0, UTMAX)
            def _(u):
                uid = s + u * NS

                @pl.when(uid < NU)
                def _():
                    pltpu.sync_copy(z_h.at[pl.ds(uid * RU, RU)],
                                    shout.at[pl.ds(uid * RU, RU)])

            plsc.subcore_barrier()

            ic_start(cid_of(0), 0)
            ic_wait(cid_of(0), 0)
            h_cp(0).start()
            ic_start(cid_of(1), 1)

            @pl.loop(0, (BSLOT - 2) // 2)
            def _(tb):
                t = tb * 2
                step(t, 0, refill=True)
                step(t + 1, 1, refill=True)
            step(BSLOT - 2, 0, refill=False)
            h_cp(1).wait()
            compute(1)
            scatter(1)

            cid = cid_of(BSLOT)

            @pl.when(cid < BCK)
            def _():
                ic_start(cid, 0)
                ic_wait(cid, 0)
                h_cp(0).start()
                h_cp(0).wait()
                compute(0)
                scatter(0)

            plsc.subcore_barrier()

            @pl.loop(0, UTMAX)
            def _(u):
                uid = s + u * NS

                @pl.when(uid < NU)
                def _():
                    pltpu.sync_copy(shout.at[pl.ds(uid * RU, RU)],
                                    o_h.at[c, pl.ds(uid * RU, RU)])

    return k(e2, coefT2, *hcols, z128)


def _proj(att):
    eye = jnp.eye(8, dtype=att.dtype)
    return (att[:, :, None] * eye[:, None, :]).reshape(F, 8)


def _gat_layer(xin_or_parts, w, p, src3, dst3, e4, z625, z128, layer2, bias_r=None):
    if layer2:
        h0, h1, h2, h3, aall, m = _tc_layer2(xin_or_parts, bias_r, w, p)
    else:
        h0, h1, h2, h3, aall, m = _tc_layer1(xin_or_parts, w, p)
    asrc_t = aall[:, :8].reshape(AROWS, 128)
    adst_t = aall[:, 8:].reshape(AROWS, 128)
    alphaT = _sc_alpha_src(asrc_t, src3)
    exT = _sc_alpha_dst_exp(adst_t, m, alphaT, dst3)
    dpart = _sc_denom(exT, dst3, z625)
    den_t = _tc_den(dpart)
    coefT = _sc_coef(den_t, exT, dst3)
    return _sc_message(e4, coefT.reshape(BCK, 8, 128), (h0, h1, h2, h3), z128)


def kernel(x, edge_index, batch, W1, att_src1, att_dst1, bias1,
           W2, att_src2, att_dst2, bias2, lin_w, lin_b):
    src3 = edge_index[0].reshape(NCK, 2, 128)
    dst3 = edge_index[1].reshape(NCK, 2, 128)
    batch3 = batch.reshape(NBLK, 1, BN)
    z625 = jnp.zeros((AROWS, 128), _f32)
    z128 = jnp.zeros((N, 128), _f32)
    p1 = jnp.concatenate([_proj(att_src1), _proj(att_dst1)], axis=1)
    p2 = jnp.concatenate([_proj(att_src2), _proj(att_dst2)], axis=1)
    b1r = bias1.reshape(1, F)
    b2r = bias2.reshape(1, F)
    lbr = lin_b.reshape(1, NCLS)

    e4 = jnp.concatenate([edge_index[0].reshape(BCK, 1, CB),
                          edge_index[1].reshape(BCK, 1, CB)], axis=1)
    o1 = _gat_layer(x, W1, p1, src3, dst3, e4, z625, z128, layer2=False)
    o2 = _gat_layer(o1, W2, p2, src3, dst3, e4, z625, z128, layer2=True,
                    bias_r=b1r)
    return _tc_pool(o2, b2r, batch3, lin_w, lbr)


# speedup vs baseline: 1.1896x; 1.1896x over previous
import dataclasses
import functools

import jax
import jax.numpy as jnp
from jax import lax
from jax.experimental import pallas as pl
from jax.experimental.pallas import tpu as pltpu
from jax.experimental.pallas import tpu_sc as plsc

N = 10000
E = 320000
DIN = 128
F = 512
NCLS = 40
NG = 64
NC = 2
NS = 16
NW = NC * NS
CE = 256
NCK = E // CE
TMAX = -(-NCK // NW)
RU = 80
NU = N // RU
UTMAX = -(-NU // NS)
AROWS = N * 8 // 128
NBLK = 10
BN = N // NBLK

_f32 = jnp.float32
_i32 = jnp.int32


def _k1_common(xb, w_ref, p_ref, h0, h1, h2, h3, aall, m, i):
    h = jnp.dot(xb, w_ref[...], preferred_element_type=_f32)
    a = jnp.dot(h, p_ref[...], preferred_element_type=_f32)
    for k, href in enumerate((h0, h1, h2, h3)):
        href[...] = h[:, k * 128:(k + 1) * 128]
    aall[...] = a
    bm = jnp.broadcast_to(jnp.max(a, axis=0, keepdims=True), (8, 16))

    @pl.when(i == 0)
    def _():
        m[...] = bm

    @pl.when(i != 0)
    def _():
        m[...] = jnp.maximum(m[...], bm)


def _k1a_body(x_ref, w_ref, p_ref, h0, h1, h2, h3, aall, m):
    _k1_common(x_ref[...], w_ref, p_ref, h0, h1, h2, h3, aall, m,
               pl.program_id(0))


def _k1b_body(o0, o1, o2, o3, b_ref, w_ref, p_ref, h0, h1, h2, h3, aall, m):
    parts = []
    for k, oref in enumerate((o0, o1, o2, o3)):
        ob = oref[...]
        parts.append(jnp.maximum(ob[0] + ob[1] + b_ref[:, k * 128:(k + 1) * 128], 0.0))
    _k1_common(jnp.concatenate(parts, axis=1), w_ref, p_ref,
               h0, h1, h2, h3, aall, m, pl.program_id(0))


def _k1_outs():
    shapes = [jax.ShapeDtypeStruct((N, 128), _f32) for _ in range(4)]
    shapes += [jax.ShapeDtypeStruct((N, 16), _f32),
               jax.ShapeDtypeStruct((8, 16), _f32)]
    specs = [pl.BlockSpec((BN, 128), lambda i: (i, 0)) for _ in range(4)]
    specs += [pl.BlockSpec((BN, 16), lambda i: (i, 0)),
              pl.BlockSpec((8, 16), lambda i: (0, 0))]
    return shapes, specs


def _tc_layer1(x, w, p):
    shapes, specs = _k1_outs()
    return pl.pallas_call(
        _k1a_body,
        grid=(NBLK,),
        in_specs=[
            pl.BlockSpec((BN, DIN), lambda i: (i, 0)),
            pl.BlockSpec((DIN, F), lambda i: (0, 0)),
            pl.BlockSpec((F, 16), lambda i: (0, 0)),
        ],
        out_specs=specs,
        out_shape=shapes,
    )(x, w, p)


def _tc_layer2(o_parts, bias_r, w, p):
    shapes, specs = _k1_outs()
    return pl.pallas_call(
        _k1b_body,
        grid=(NBLK,),
        in_specs=[pl.BlockSpec((2, BN, 128), lambda i: (0, i, 0))] * 4 + [
            pl.BlockSpec((1, F), lambda i: (0, 0)),
            pl.BlockSpec((F, F), lambda i: (0, 0)),
            pl.BlockSpec((F, 16), lambda i: (0, 0)),
        ],
        out_specs=specs,
        out_shape=shapes,
    )(*o_parts, bias_r, w, p)


def _den_body(d_ref, o_ref):
    o_ref[...] = jnp.sum(d_ref[...], axis=0) + 1e-16


def _tc_den(dpart):
    return pl.pallas_call(
        _den_body,
        out_shape=jax.ShapeDtypeStruct((AROWS, 128), _f32),
    )(dpart)


def _pool_body(o0, o1, o2, o3, b_ref, bat_ref, lw_ref, lb_ref, probs_ref,
               pool_scr, cnt_scr):
    i = pl.program_id(0)

    @pl.when(i == 0)
    def _():
        pool_scr[...] = jnp.zeros((NG, F), _f32)
        cnt_scr[...] = jnp.zeros((NG, 128), _f32)

    parts = []
    for k, oref in enumerate((o0, o1, o2, o3)):
        ob = oref[...]
        parts.append(jnp.maximum(ob[0] + ob[1] + b_ref[:, k * 128:(k + 1) * 128], 0.0))
    h2 = jnp.concatenate(parts, axis=1)
    bb = bat_ref[0]
    oh = (lax.broadcasted_iota(_i32, (NG, BN), 0) == bb).astype(_f32)
    pool_scr[...] += jnp.dot(oh, h2, preferred_element_type=_f32)
    cnt_scr[...] += jnp.broadcast_to(jnp.sum(oh, axis=1, keepdims=True), (NG, 128))

    @pl.when(i == NBLK - 1)
    def _():
        cnt = jnp.maximum(cnt_scr[:, :1], 1.0)
        pooled = pool_scr[...] / cnt
        logits = jnp.dot(pooled, lw_ref[...], preferred_element_type=_f32) + lb_ref[...]
        mx = jnp.max(logits, axis=1, keepdims=True)
        e = jnp.exp(logits - mx)
        probs_ref[...] = e / jnp.sum(e, axis=1, keepdims=True)


def _tc_pool(o_parts, bias_r, batch3, lin_w, lin_br):
    return pl.pallas_call(
        _pool_body,
        grid=(NBLK,),
        in_specs=[pl.BlockSpec((2, BN, 128), lambda i: (0, i, 0))] * 4 + [
            pl.BlockSpec((1, F), lambda i: (0, 0)),
            pl.BlockSpec((1, 1, BN), lambda i: (i, 0, 0)),
            pl.BlockSpec((F, NCLS), lambda i: (0, 0)),
            pl.BlockSpec((1, NCLS), lambda i: (0, 0)),
        ],
        out_specs=pl.BlockSpec((NG, NCLS), lambda i: (0, 0)),
        out_shape=jax.ShapeDtypeStruct((NG, NCLS), _f32),
        scratch_shapes=[
            pltpu.VMEM((NG, F), _f32),
            pltpu.VMEM((NG, 128), _f32),
        ],
    )(*o_parts, bias_r, batch3, lin_w, lin_br)


_MESH = plsc.VectorSubcoreMesh(core_axis_name="c", subcore_axis_name="s")
_IOTA = lambda: lax.iota(_i32, 16)

_SC_PARAMS = pltpu.CompilerParams()
if "needs_layout_passes" in pltpu.CompilerParams.__dataclass_fields__:
    _SC_PARAMS = dataclasses.replace(_SC_PARAMS, needs_layout_passes=False)


def _worker_chunks(w, body):
    @pl.loop(0, TMAX)
    def _(t):
        cid = w + t * NW

        @pl.when(cid < NCK)
        def _():
            body(cid)


def _sc_alpha_src(asrc_t, src3):
    @functools.partial(
        pl.kernel,
        out_type=jax.ShapeDtypeStruct((NCK, 16, 128), _f32),
        mesh=_MESH,
        compiler_params=_SC_PARAMS,
        scratch_types=[
            pltpu.VMEM((AROWS, 128), _f32),
            pltpu.VMEM((2, 128), _i32),
            pltpu.VMEM((16, 128), _f32),
        ],
    )
    def k(at_h, src_h, out_h, atab, srcv, obuf):
        c = lax.axis_index("c")
        s = lax.axis_index("s")
        w = c * NS + s
        pltpu.sync_copy(at_h, atab)

        def chunk(cid):
            pltpu.sync_copy(src_h.at[cid], srcv)
            for g in range(16):
                sv = srcv[g // 8, pl.ds((g % 8) * 16, 16)]
                for h in range(8):
                    idx = sv * 8 + h
                    v = plsc.load_gather(
                        atab, [lax.shift_right_logical(idx, 7), idx & 127])
                    obuf[g, pl.ds(h * 16, 16)] = v
            pltpu.sync_copy(obuf, out_h.at[cid])

        _worker_chunks(w, chunk)

    return k(asrc_t, src3)


def _sc_alpha_dst_exp(adst_t, m, alphaT, dst3):
    @functools.partial(
        pl.kernel,
        out_type=jax.ShapeDtypeStruct((NCK, 16, 128), _f32),
        mesh=_MESH,
        compiler_params=_SC_PARAMS,
        scratch_types=[
            pltpu.VMEM((AROWS, 128), _f32),
            pltpu.VMEM((2, 128), _i32),
            pltpu.VMEM((16, 128), _f32),
            pltpu.VMEM((16, 128), _f32),
            pltpu.VMEM((8, 16), _f32),
        ],
    )
    def k(at_h, m_h, al_h, dst_h, out_h, atab, dstv, abuf, ebuf, mb):
        c = lax.axis_index("c")
        s = lax.axis_index("s")
        w = c * NS + s
        pltpu.sync_copy(at_h, atab)
        pltpu.sync_copy(m_h, mb)
        msp = []
        for h in range(8):
            ms = plsc.load_gather(mb, [jnp.full((16,), 0, _i32),
                                       jnp.full((16,), h, _i32)])
            md = plsc.load_gather(mb, [jnp.full((16,), 0, _i32),
                                       jnp.full((16,), h + 8, _i32)])
            msp.append(ms + md)

        def chunk(cid):
            pltpu.sync_copy(dst_h.at[cid], dstv)
            pltpu.sync_copy(al_h.at[cid], abuf)
            for g in range(16):
                dv = dstv[g // 8, pl.ds((g % 8) * 16, 16)]
                for h in range(8):
                    idx = dv * 8 + h
                    d = plsc.load_gather(
                        atab, [lax.shift_right_logical(idx, 7), idx & 127])
                    a = abuf[g, pl.ds(h * 16, 16)] + d
                    al = jnp.maximum(a, 0.2 * a) - msp[h]
                    ebuf[g, pl.ds(h * 16, 16)] = jnp.exp(al)
            pltpu.sync_copy(ebuf, out_h.at[cid])

        _worker_chunks(w, chunk)

    return k(adst_t, m, alphaT, dst3)


def _sc_denom(exT, dst3, z625):
    @functools.partial(
        pl.kernel,
        out_type=jax.ShapeDtypeStruct((NW, AROWS, 128), _f32),
        mesh=_MESH,
        compiler_params=_SC_PARAMS,
        scratch_types=[
            pltpu.VMEM((AROWS, 128), _f32),
            pltpu.VMEM((2, 128), _i32),
            pltpu.VMEM((16, 128), _f32),
        ],
    )
    def k(ex_h, dst_h, z_h, out_h, den, dstv, ebuf):
        c = lax.axis_index("c")
        s = lax.axis_index("s")
        w = c * NS + s
        pltpu.sync_copy(z_h, den)
        lanes = _IOTA()
        mask = lanes < 8
        hsel = lanes & 7

        def chunk(cid):
            pltpu.sync_copy(dst_h.at[cid], dstv)
            pltpu.sync_copy(ex_h.at[cid], ebuf)

            @pl.loop(0, CE)
            def _(i):
                dsp = plsc.load_gather(
                    dstv, [jnp.full((16,), lax.shift_right_logical(i, 7), _i32),
                           jnp.full((16,), i & 127, _i32)])
                exr = plsc.load_gather(
                    ebuf, [jnp.full((16,), lax.shift_right_logical(i, 4), _i32),
                           (hsel * 16 + (i & 15))])
                idx = dsp * 8 + hsel
                plsc.addupdate_scatter(
                    den, [lax.shift_right_logical(idx, 7), idx & 127],
                    exr, mask=mask)

        _worker_chunks(w, chunk)
        pltpu.sync_copy(den, out_h.at[w])

    return k(exT, dst3, z625)


def _sc_coef(den_t, exT, dst3):
    @functools.partial(
        pl.kernel,
        out_type=jax.ShapeDtypeStruct((NCK, 16, 128), _f32),
        mesh=_MESH,
        compiler_params=_SC_PARAMS,
        scratch_types=[
            pltpu.VMEM((AROWS, 128), _f32),
            pltpu.VMEM((2, 128), _i32),
            pltpu.VMEM((16, 128), _f32),
            pltpu.VMEM((16, 128), _f32),
        ],
    )
    def k(dt_h, ex_h, dst_h, out_h, dtab, dstv, ebuf, cbuf):
        c = lax.axis_index("c")
        s = lax.axis_index("s")
        w = c * NS + s
        pltpu.sync_copy(dt_h, dtab)

        def chunk(cid):
            pltpu.sync_copy(dst_h.at[cid], dstv)
            pltpu.sync_copy(ex_h.at[cid], ebuf)
            for g in range(16):
                dv = dstv[g // 8, pl.ds((g % 8) * 16, 16)]
                for h in range(8):
                    idx = dv * 8 + h
                    den = plsc.load_gather(
                        dtab, [lax.shift_right_logical(idx, 7), idx & 127])
                    cbuf[g, pl.ds(h * 16, 16)] = \
                        ebuf[g, pl.ds(h * 16, 16)] / den
            pltpu.sync_copy(cbuf, out_h.at[cid])

        _worker_chunks(w, chunk)

    return k(den_t, exT, dst3)


CB = 256
BCK = E // CB


def _sc_message(e2, coefT2, hcols, z128):
    @functools.partial(
        pl.kernel,
        out_type=[jax.ShapeDtypeStruct((2, N, 128), _f32) for _ in range(4)],
        mesh=_MESH,
        compiler_params=_SC_PARAMS,
        scratch_types=[
            pltpu.VMEM((4, 128), _i32),
            pltpu.VMEM((16, 128), _f32),
            pltpu.VMEM((CB, 128), _f32),
            pltpu.VMEM_SHARED((N, 128), _f32),
            pltpu.SemaphoreType.DMA,
            pltpu.SemaphoreType.DMA,
        ],
    )
    def k(e2_h, co_h, h0_h, h1_h, h2_h, h3_h, z_h,
          o0_h, o1_h, o2_h, o3_h,
          ebuf, cbuf, hbuf, shout, sem1, sem2):
        c = lax.axis_index("c")
        s = lax.axis_index("s")
        w = c * NS + s
        h_refs = (h0_h, h1_h, h2_h, h3_h)
        o_refs = (o0_h, o1_h, o2_h, o3_h)
        for kcol in range(4):
            h_h = h_refs[kcol]
            o_h = o_refs[kcol]

            @pl.loop(0, UTMAX)
            def _(u):
                uid = s + u * NS

                @pl.when(uid < NU)
                def _():
                    pltpu.sync_copy(z_h.at[pl.ds(uid * RU, RU)],
                                    shout.at[pl.ds(uid * RU, RU)])

            plsc.subcore_barrier()

            def chunk(cid):
                pltpu.sync_copy(e2_h.at[cid], ebuf)
                pltpu.sync_copy(co_h.at[cid], cbuf)
                cp1 = pltpu.async_copy(
                    h_h.at[ebuf.at[0]], hbuf.at[pl.ds(0, 128)], sem1)
                cp2 = pltpu.async_copy(
                    h_h.at[ebuf.at[1]], hbuf.at[pl.ds(128, 128)], sem2)
                cp1.wait()
                cp2.wait()

                @pl.loop(0, 16)
                def _(g):
                    gsp = jnp.full((16,), g, _i32)

                    @pl.loop(0, 16, step=2)
                    def _(l):
                        for dl in range(2):
                            c0 = jnp.full((16,), 32 * kcol + l + dl, _i32)
                            b0 = plsc.load_gather(cbuf, [gsp, c0])
                            b1 = plsc.load_gather(cbuf, [gsp, c0 + 16])
                            i = g * 16 + l + dl
                            for j2 in range(8):
                                sel = b0 if j2 < 4 else b1
                                sl = pl.ds(j2 * 16, 16)
                                hbuf[i, sl] = hbuf[i, sl] * sel

                for j in range(2):
                    pltpu.sync_copy(hbuf.at[pl.ds(j * 128, 128)],
                                    shout.at[ebuf.at[2 + j]], add=True)

            _worker_chunks(w, chunk)
            plsc.subcore_barrier()

            @pl.loop(0, UTMAX)
            def _(u):
                uid = s + u * NS

                @pl.when(uid < NU)
                def _():
                    pltpu.sync_copy(shout.at[pl.ds(uid * RU, RU)],
                                    o_h.at[c, pl.ds(uid * RU, RU)])

    return k(e2, coefT2, *hcols, z128)


def _proj(att):
    eye = jnp.eye(8, dtype=att.dtype)
    return (att[:, :, None] * eye[:, None, :]).reshape(F, 8)


def _gat_layer(xin_or_parts, w, p, src3, dst3, e4, z625, z128, layer2, bias_r=None):
    if layer2:
        h0, h1, h2, h3, aall, m = _tc_layer2(xin_or_parts, bias_r, w, p)
    else:
        h0, h1, h2, h3, aall, m = _tc_layer1(xin_or_parts, w, p)
    asrc_t = aall[:, :8].reshape(AROWS, 128)
    adst_t = aall[:, 8:].reshape(AROWS, 128)
    alphaT = _sc_alpha_src(asrc_t, src3)
    exT = _sc_alpha_dst_exp(adst_t, m, alphaT, dst3)
    dpart = _sc_denom(exT, dst3, z625)
    den_t = _tc_den(dpart)
    coefT = _sc_coef(den_t, exT, dst3)
    return _sc_message(e4, coefT, (h0, h1, h2, h3), z128)


def kernel(x, edge_index, batch, W1, att_src1, att_dst1, bias1,
           W2, att_src2, att_dst2, bias2, lin_w, lin_b):
    src3 = edge_index[0].reshape(NCK, 2, 128)
    dst3 = edge_index[1].reshape(NCK, 2, 128)
    batch3 = batch.reshape(NBLK, 1, BN)
    z625 = jnp.zeros((AROWS, 128), _f32)
    z128 = jnp.zeros((N, 128), _f32)
    p1 = jnp.concatenate([_proj(att_src1), _proj(att_dst1)], axis=1)
    p2 = jnp.concatenate([_proj(att_src2), _proj(att_dst2)], axis=1)
    b1r = bias1.reshape(1, F)
    b2r = bias2.reshape(1, F)
    lbr = lin_b.reshape(1, NCLS)

    e4 = jnp.concatenate([src3, dst3], axis=1)
    o1 = _gat_layer(x, W1, p1, src3, dst3, e4, z625, z128, layer2=False)
    o2 = _gat_layer(o1, W2, p2, src3, dst3, e4, z625, z128, layer2=True,
                    bias_r=b1r)
    return _tc_pool(o2, b2r, batch3, lin_w, lbr)
